# baseline (speedup 1.0000x reference)
import functools

import jax
import jax.numpy as jnp
from jax import lax
from jax.experimental import pallas as pl
from jax.experimental.pallas import tpu as pltpu
from jax.experimental.pallas import tpu_sc as plsc

N = 10000
E = 320000
F = 128
ALPHA = 0.2

NC = 2
NS = 16
NW = NC * NS
EPT = E // NW
CH = 80
NCHUNK = EPT // CH
NP = 10240
RPT = NP // NS
DR = NP // F

BLK = 400
BLK1 = 256


def _phase1_body(x_ref, w_ref, a_ref, h_ref, hl_ref, hr_ref, z2_ref, zd_ref):
    i = pl.program_id(0)
    h = jnp.dot(x_ref[...], w_ref[...], preferred_element_type=jnp.float32)
    h_ref[...] = h
    hlr = jnp.dot(h, a_ref[...], preferred_element_type=jnp.float32)
    hl_ref[pl.ds(i * BLK1, BLK1)] = hlr[:, 0]
    hr_ref[pl.ds(i * BLK1, BLK1)] = hlr[:, 1]

    @pl.when(i == 0)
    def _():
        z2_ref[...] = jnp.zeros((RPT, F), jnp.float32)
        zd_ref[...] = jnp.zeros((DR, F), jnp.float32)


def _phase1(x, W, A):
    return pl.pallas_call(
        _phase1_body,
        grid=(NP // BLK1,),
        in_specs=[
            pl.BlockSpec((BLK1, F), lambda i: (i, 0)),
            pl.BlockSpec((F, F), lambda i: (0, 0)),
            pl.BlockSpec((F, 8), lambda i: (0, 0)),
        ],
        out_specs=[
            pl.BlockSpec((BLK1, F), lambda i: (i, 0)),
            pl.BlockSpec((NP,), lambda i: (0,)),
            pl.BlockSpec((NP,), lambda i: (0,)),
            pl.BlockSpec((RPT, F), lambda i: (0, 0)),
            pl.BlockSpec((DR, F), lambda i: (0, 0)),
        ],
        out_shape=[
            jax.ShapeDtypeStruct((NP, F), jnp.float32),
            jax.ShapeDtypeStruct((NP,), jnp.float32),
            jax.ShapeDtypeStruct((NP,), jnp.float32),
            jax.ShapeDtypeStruct((RPT, F), jnp.float32),
            jax.ShapeDtypeStruct((DR, F), jnp.float32),
        ],
    )(x, W, A)


def _lane_bcast(vec, j):
    idx = jnp.full((16, 1), j, jnp.int32)
    dnums = lax.GatherDimensionNumbers(
        offset_dims=(), collapsed_slice_dims=(0,), start_index_map=(0,))
    return lax.gather(vec, idx, dnums, (1,),
                      mode=lax.GatherScatterMode.PROMISE_IN_BOUNDS)


_sc_mesh = plsc.VectorSubcoreMesh(core_axis_name="c", subcore_axis_name="s")


@functools.partial(
    pl.kernel,
    mesh=_sc_mesh,
    out_type=[
        jax.ShapeDtypeStruct((E,), jnp.float32),
        jax.ShapeDtypeStruct((NC * DR, F), jnp.float32),
    ],
    scratch_types=[
        pltpu.VMEM((NP,), jnp.float32),
        pltpu.VMEM((NP,), jnp.float32),
        pltpu.VMEM((DR, F), jnp.float32),
        pltpu.VMEM((EPT,), jnp.int32),
        pltpu.VMEM((EPT,), jnp.int32),
        pltpu.VMEM((EPT,), jnp.float32),
        pltpu.VMEM((DR,), jnp.int32),
        pltpu.VMEM_SHARED((DR, F), jnp.float32),
    ],
    compiler_params=pltpu.CompilerParams(needs_layout_passes=False),
)
def _weight_kernel(rowi, coli, hl, hr, zd, w_out, dend,
                   hl_v, hr_v, den_v, row_t, col_t, w_all, idx80, den_sh):
    c = lax.axis_index("c")
    s = lax.axis_index("s")
    wid = c * NS + s
    ebase = pl.multiple_of(wid * EPT, 16)
    pltpu.sync_copy(hl, hl_v)
    pltpu.sync_copy(hr, hr_v)
    pltpu.sync_copy(zd, den_v)
    pltpu.sync_copy(rowi.at[pl.ds(ebase, EPT)], row_t)
    pltpu.sync_copy(coli.at[pl.ds(ebase, EPT)], col_t)

    @pl.when(s == 0)
    def _():
        pltpu.sync_copy(zd, den_sh)

    lane = lax.iota(jnp.int32, 16)
    for i in range(DR // 16):
        idx80[pl.ds(i * 16, 16)] = lane + i * 16

    def body(i, carry):
        sl = pl.ds(pl.multiple_of(i * 16, 16), 16)
        rv = row_t[sl]
        cv = col_t[sl]
        e = plsc.load_gather(hl_v, [rv]) + plsc.load_gather(hr_v, [cv])
        e = jnp.where(e > 0, e, ALPHA * e)
        w = jnp.exp(e)
        w_all[sl] = w
        rhi = lax.shift_right_logical(rv, 7)
        rlo = lax.bitwise_and(rv, 127)
        for j in range(16):
            plsc.addupdate_scatter(den_v, [rhi, rlo], w, mask=lane == j)
        return carry

    lax.fori_loop(0, EPT // 16, body, 0)
    pltpu.sync_copy(w_all, w_out.at[pl.ds(ebase, EPT)])
    plsc.subcore_barrier()
    pltpu.sync_copy(den_v, den_sh.at[idx80], add=True)
    plsc.subcore_barrier()

    @pl.when(s == 0)
    def _():
        pltpu.sync_copy(den_sh, dend.at[pl.ds(c * DR, DR)])


@functools.partial(
    pl.kernel,
    mesh=_sc_mesh,
    out_type=jax.ShapeDtypeStruct((NC * NP, F), jnp.float32),
    scratch_types=[
        pltpu.VMEM((3, CH, F), jnp.float32),
        pltpu.VMEM((2, CH), jnp.int32),
        pltpu.VMEM((2, CH), jnp.int32),
        pltpu.VMEM((2, CH), jnp.float32),
        pltpu.VMEM((2, CH), jnp.int32),
        pltpu.VMEM_SHARED((NP, F), jnp.float32),
        pltpu.SemaphoreType.DMA((3,)),
        pltpu.SemaphoreType.DMA((2,)),
        pltpu.SemaphoreType.DMA,
    ],
    compiler_params=pltpu.CompilerParams(needs_layout_passes=False),
)
def _spmm_kernel(h, rowi, coli, w_in, zros, out,
                 rows3, rc, ic, wc, row_s, acc, sem_g, sem_i, sem_s):
    c = lax.axis_index("c")
    s = lax.axis_index("s")
    wid = c * NS + s
    ebase = wid * EPT

    pltpu.sync_copy(zros, acc.at[pl.ds(s * RPT, RPT)])
    plsc.subcore_barrier()

    def esl(k):
        return pl.ds(pl.multiple_of(ebase + k * CH, 16), CH)

    def idx_load_sync(k, b):
        pltpu.sync_copy(rowi.at[esl(k)], rc.at[b])
        pltpu.sync_copy(coli.at[esl(k)], ic.at[b])
        pltpu.sync_copy(w_in.at[esl(k)], wc.at[b])

    def idx_prefetch(k, b):
        pltpu.async_copy(rowi.at[esl(k)], rc.at[b], sem_i.at[b])
        pltpu.async_copy(coli.at[esl(k)], ic.at[b], sem_i.at[b])
        pltpu.async_copy(w_in.at[esl(k)], wc.at[b], sem_i.at[b])

    def idx_wait(k, b):
        pltpu.make_async_copy(rowi.at[esl(k)], rc.at[b], sem_i.at[b]).wait()
        pltpu.make_async_copy(coli.at[esl(k)], ic.at[b], sem_i.at[b]).wait()
        pltpu.make_async_copy(w_in.at[esl(k)], wc.at[b], sem_i.at[b]).wait()

    def gather_start(k, g, b):
        pltpu.async_copy(h.at[ic.at[b]], rows3.at[g], sem_g.at[g])

    def gather_wait(k, g, b):
        pltpu.make_async_copy(h.at[ic.at[b]], rows3.at[g], sem_g.at[g]).wait()

    def scatter_start(g, b):
        pltpu.async_copy(rows3.at[g], acc.at[row_s.at[b]], sem_s, add=True)

    def scatter_wait(g, b):
        pltpu.make_async_copy(rows3.at[g], acc.at[row_s.at[b]], sem_s).wait()

    def scale_and_stage(g, b):
        for i in range(CH // 16):
            sl = pl.ds(i * 16, 16)
            wv = wc[b, sl]
            row_s[b, sl] = rc[b, sl]
            for j in range(16):
                wspl = _lane_bcast(wv, j)
                eidx = i * 16 + j
                for r in range(F // 16):
                    fsl = pl.ds(r * 16, 16)
                    rows3[g, eidx, fsl] = rows3[g, eidx, fsl] * wspl

    idx_load_sync(0, 0)
    idx_load_sync(1, 1)
    gather_start(0, 0, 0)
    gather_start(1, 1, 1)
    gather_wait(0, 0, 0)
    scale_and_stage(0, 0)
    scatter_start(0, 0)
    idx_prefetch(2, 0)
    idx_wait(2, 0)
    gather_start(2, 2, 0)
    gather_wait(1, 1, 1)
    scale_and_stage(1, 1)
    scatter_start(1, 1)
    idx_prefetch(3, 1)

    def step(k, carry):
        b = lax.rem(k, 2)
        g = lax.rem(k, 3)
        bn = lax.rem(k + 1, 2)
        gn = lax.rem(k + 1, 3)
        scatter_wait(gn, b)

        @pl.when(k + 1 < NCHUNK)
        def _():
            idx_wait(k + 1, bn)
            gather_start(k + 1, gn, bn)

        gather_wait(k, g, b)
        scale_and_stage(g, b)
        scatter_start(g, b)

        @pl.when(k + 2 < NCHUNK)
        def _():
            idx_prefetch(k + 2, b)

        return carry

    lax.fori_loop(2, NCHUNK, step, 0)
    scatter_wait((NCHUNK - 2) % 3, (NCHUNK - 2) % 2)
    scatter_wait((NCHUNK - 1) % 3, (NCHUNK - 1) % 2)

    plsc.subcore_barrier()
    pltpu.sync_copy(acc.at[pl.ds(s * RPT, RPT)],
                    out.at[pl.ds(c * NP + s * RPT, RPT)])


def _phase3_body(p_ref, d_ref, o_ref):
    ss = p_ref[0] + p_ref[1]
    den = jnp.sum(d_ref[...], axis=1) + 1e-16
    o_ref[...] = ss / den[:, None]


def _phase3(partial, denom):
    return pl.pallas_call(
        _phase3_body,
        grid=(N // BLK,),
        in_specs=[
            pl.BlockSpec((NC, BLK, F), lambda i: (0, i, 0)),
            pl.BlockSpec((BLK, NC), lambda i: (i, 0)),
        ],
        out_specs=pl.BlockSpec((BLK, F), lambda i: (i, 0)),
        out_shape=jax.ShapeDtypeStruct((N, F), jnp.float32),
    )(partial, denom)


def kernel(x, edge_index, W, a_l, a_r):
    al = a_l.reshape(F)
    ar = a_r.reshape(F)
    A = jnp.zeros((F, 8), jnp.float32).at[:, 0].set(al).at[:, 1].set(ar)
    h, hl, hr, zros2, zrosd2 = _phase1(x, W, A)
    row = edge_index[0]
    col = edge_index[1]
    w_e, dend = _weight_kernel(row, col, hl, hr, zrosd2)
    partial = _spmm_kernel(h, row, col, w_e, zros2)
    return _phase3(partial.reshape(NC, NP, F), dend.reshape(NC, NP).T)


# speedup vs baseline: 1.1413x; 1.1413x over previous
import functools

import jax
import jax.numpy as jnp
from jax import lax
from jax.experimental import pallas as pl
from jax.experimental.pallas import tpu as pltpu
from jax.experimental.pallas import tpu_sc as plsc

N = 10000
E = 320000
F = 128
ALPHA = 0.2

NC = 2
NS = 16
NW = NC * NS
EPT = E // NW
CH = 80
NCHUNK = EPT // CH
NP = 10240
RPT = NP // NS
DR = NP // F

BLK = 400


def _phase1_body(x_ref, w_ref, al_ref, ar_ref,
                 h_ref, hl_ref, hr_ref, z2_ref, zd_ref):
    h = jnp.dot(x_ref[...], w_ref[...], preferred_element_type=jnp.float32)
    h_ref[...] = h
    hl_ref[...] = jnp.sum(h * al_ref[0, :][None, :], axis=1)
    hr_ref[...] = jnp.sum(h * ar_ref[0, :][None, :], axis=1)
    z2_ref[...] = jnp.zeros((RPT, F), jnp.float32)
    zd_ref[...] = jnp.zeros((DR, F), jnp.float32)


def _phase1(x, W, al, ar):
    return pl.pallas_call(
        _phase1_body,
        out_shape=[
            jax.ShapeDtypeStruct((N, F), jnp.float32),
            jax.ShapeDtypeStruct((N,), jnp.float32),
            jax.ShapeDtypeStruct((N,), jnp.float32),
            jax.ShapeDtypeStruct((RPT, F), jnp.float32),
            jax.ShapeDtypeStruct((DR, F), jnp.float32),
        ],
    )(x, W, al, ar)


def _lane_bcast(vec, j):
    idx = jnp.full((16, 1), j, jnp.int32)
    dnums = lax.GatherDimensionNumbers(
        offset_dims=(), collapsed_slice_dims=(0,), start_index_map=(0,))
    return lax.gather(vec, idx, dnums, (1,),
                      mode=lax.GatherScatterMode.PROMISE_IN_BOUNDS)


_sc_mesh = plsc.VectorSubcoreMesh(core_axis_name="c", subcore_axis_name="s")


@functools.partial(
    pl.kernel,
    mesh=_sc_mesh,
    out_type=[
        jax.ShapeDtypeStruct((E,), jnp.float32),
        jax.ShapeDtypeStruct((NC * DR, F), jnp.float32),
    ],
    scratch_types=[
        pltpu.VMEM((N,), jnp.float32),
        pltpu.VMEM((N,), jnp.float32),
        pltpu.VMEM((DR, F), jnp.float32),
        pltpu.VMEM((EPT,), jnp.int32),
        pltpu.VMEM((EPT,), jnp.int32),
        pltpu.VMEM((EPT,), jnp.float32),
        pltpu.VMEM((DR,), jnp.int32),
        pltpu.VMEM_SHARED((DR, F), jnp.float32),
    ],
    compiler_params=pltpu.CompilerParams(needs_layout_passes=False),
)
def _weight_kernel(eif, hl, hr, zd, w_out, dend,
                   hl_v, hr_v, den_v, row_t, col_t, w_all, idx80, den_sh):
    c = lax.axis_index("c")
    s = lax.axis_index("s")
    wid = c * NS + s
    ebase = pl.multiple_of(wid * EPT, 16)
    pltpu.sync_copy(hl, hl_v)
    pltpu.sync_copy(hr, hr_v)
    pltpu.sync_copy(zd, den_v)
    pltpu.sync_copy(eif.at[pl.ds(ebase, EPT)], row_t)
    pltpu.sync_copy(eif.at[pl.ds(E + ebase, EPT)], col_t)

    @pl.when(s == 0)
    def _():
        pltpu.sync_copy(zd, den_sh)

    lane = lax.iota(jnp.int32, 16)
    for i in range(DR // 16):
        idx80[pl.ds(i * 16, 16)] = lane + i * 16

    def body(i, carry):
        sl = pl.ds(pl.multiple_of(i * 16, 16), 16)
        rv = row_t[sl]
        cv = col_t[sl]
        e = plsc.load_gather(hl_v, [rv]) + plsc.load_gather(hr_v, [cv])
        e = jnp.where(e > 0, e, ALPHA * e)
        w = jnp.exp(e)
        w_all[sl] = w
        rhi = lax.shift_right_logical(rv, 7)
        rlo = lax.bitwise_and(rv, 127)
        for j in range(16):
            plsc.addupdate_scatter(den_v, [rhi, rlo], w, mask=lane == j)
        return carry

    lax.fori_loop(0, EPT // 16, body, 0)
    pltpu.sync_copy(w_all, w_out.at[pl.ds(ebase, EPT)])
    plsc.subcore_barrier()
    pltpu.sync_copy(den_v, den_sh.at[idx80], add=True)
    plsc.subcore_barrier()

    @pl.when(s == 0)
    def _():
        pltpu.sync_copy(den_sh, dend.at[pl.ds(c * DR, DR)])


@functools.partial(
    pl.kernel,
    mesh=_sc_mesh,
    out_type=jax.ShapeDtypeStruct((NC * NP, F), jnp.float32),
    scratch_types=[
        pltpu.VMEM((3, CH, F), jnp.float32),
        pltpu.VMEM((2, CH), jnp.int32),
        pltpu.VMEM((2, CH), jnp.int32),
        pltpu.VMEM((2, CH), jnp.float32),
        pltpu.VMEM((2, CH), jnp.int32),
        pltpu.VMEM_SHARED((NP, F), jnp.float32),
        pltpu.SemaphoreType.DMA((3,)),
        pltpu.SemaphoreType.DMA((2,)),
        pltpu.SemaphoreType.DMA,
    ],
    compiler_params=pltpu.CompilerParams(needs_layout_passes=False),
)
def _spmm_kernel(h, eif, w_in, zros, out,
                 rows3, rc, ic, wc, row_s, acc, sem_g, sem_i, sem_s):
    c = lax.axis_index("c")
    s = lax.axis_index("s")
    wid = c * NS + s
    ebase = wid * EPT

    pltpu.sync_copy(zros, acc.at[pl.ds(s * RPT, RPT)])
    plsc.subcore_barrier()

    def esl(k):
        return pl.ds(pl.multiple_of(ebase + k * CH, 16), CH)

    def csl(k):
        return pl.ds(pl.multiple_of(E + ebase + k * CH, 16), CH)

    def idx_load_sync(k, b):
        pltpu.sync_copy(eif.at[esl(k)], rc.at[b])
        pltpu.sync_copy(eif.at[csl(k)], ic.at[b])
        pltpu.sync_copy(w_in.at[esl(k)], wc.at[b])

    def idx_prefetch(k, b):
        pltpu.async_copy(eif.at[esl(k)], rc.at[b], sem_i.at[b])
        pltpu.async_copy(eif.at[csl(k)], ic.at[b], sem_i.at[b])
        pltpu.async_copy(w_in.at[esl(k)], wc.at[b], sem_i.at[b])

    def idx_wait(k, b):
        pltpu.make_async_copy(eif.at[esl(k)], rc.at[b], sem_i.at[b]).wait()
        pltpu.make_async_copy(eif.at[csl(k)], ic.at[b], sem_i.at[b]).wait()
        pltpu.make_async_copy(w_in.at[esl(k)], wc.at[b], sem_i.at[b]).wait()

    def gather_start(k, g, b):
        pltpu.async_copy(h.at[ic.at[b]], rows3.at[g], sem_g.at[g])

    def gather_wait(k, g, b):
        pltpu.make_async_copy(h.at[ic.at[b]], rows3.at[g], sem_g.at[g]).wait()

    def scatter_start(g, b):
        pltpu.async_copy(rows3.at[g], acc.at[row_s.at[b]], sem_s, add=True)

    def scatter_wait(g, b):
        pltpu.make_async_copy(rows3.at[g], acc.at[row_s.at[b]], sem_s).wait()

    def scale_and_stage(g, b):
        for i in range(CH // 16):
            sl = pl.ds(i * 16, 16)
            wv = wc[b, sl]
            row_s[b, sl] = rc[b, sl]
            for j in range(16):
                wspl = _lane_bcast(wv, j)
                eidx = i * 16 + j
                for r in range(F // 16):
                    fsl = pl.ds(r * 16, 16)
                    rows3[g, eidx, fsl] = rows3[g, eidx, fsl] * wspl

    idx_load_sync(0, 0)
    idx_load_sync(1, 1)
    gather_start(0, 0, 0)
    gather_start(1, 1, 1)
    gather_wait(0, 0, 0)
    scale_and_stage(0, 0)
    scatter_start(0, 0)
    idx_prefetch(2, 0)
    idx_wait(2, 0)
    gather_start(2, 2, 0)
    gather_wait(1, 1, 1)
    scale_and_stage(1, 1)
    scatter_start(1, 1)
    idx_prefetch(3, 1)

    def step(k, carry):
        b = lax.rem(k, 2)
        g = lax.rem(k, 3)
        bn = lax.rem(k + 1, 2)
        gn = lax.rem(k + 1, 3)
        scatter_wait(gn, b)

        @pl.when(k + 1 < NCHUNK)
        def _():
            idx_wait(k + 1, bn)
            gather_start(k + 1, gn, bn)

        gather_wait(k, g, b)
        scale_and_stage(g, b)
        scatter_start(g, b)

        @pl.when(k + 2 < NCHUNK)
        def _():
            idx_prefetch(k + 2, b)

        return carry

    lax.fori_loop(2, NCHUNK, step, 0)
    scatter_wait((NCHUNK - 2) % 3, (NCHUNK - 2) % 2)
    scatter_wait((NCHUNK - 1) % 3, (NCHUNK - 1) % 2)

    plsc.subcore_barrier()
    pltpu.sync_copy(acc.at[pl.ds(s * RPT, RPT)],
                    out.at[pl.ds(c * NP + s * RPT, RPT)])


def _phase3_body(p_ref, d_ref, o_ref):
    ss = p_ref[0] + p_ref[1]
    den = jnp.sum(d_ref[...], axis=1) + 1e-16
    o_ref[...] = ss / den[:, None]


def _phase3(partial, denom):
    return pl.pallas_call(
        _phase3_body,
        grid=(N // BLK,),
        in_specs=[
            pl.BlockSpec((NC, BLK, F), lambda i: (0, i, 0)),
            pl.BlockSpec((BLK, NC), lambda i: (i, 0)),
        ],
        out_specs=pl.BlockSpec((BLK, F), lambda i: (i, 0)),
        out_shape=jax.ShapeDtypeStruct((N, F), jnp.float32),
    )(partial, denom)


def kernel(x, edge_index, W, a_l, a_r):
    al = a_l.reshape(1, F)
    ar = a_r.reshape(1, F)
    h, hl, hr, zros2, zrosd2 = _phase1(x, W, al, ar)
    eif = edge_index.reshape(2 * E)
    w_e, dend = _weight_kernel(eif, hl, hr, zrosd2)
    partial = _spmm_kernel(h, eif, w_e, zros2)
    return _phase3(partial.reshape(NC, NP, F), dend.reshape(NC, NP).T)
